# baseline (device time: 20703 ns/iter reference)
import jax
import jax.numpy as jnp
from jax import lax
from jax.experimental import pallas as pl
from jax.experimental.pallas import tpu as pltpu

N_DEV = 16
N_TOK = 512
D_IN = 256
D_OUT = 512
N_EXP = 32
E_LOC = N_EXP // N_DEV
CAP = 12
BLK = N_TOK // N_DEV


def kernel(x, router_W, route_idx, expert_W):
    del router_W

    def body(x_ref, idx_ref, w_ref, out_ref, partial_ref, comm_ref,
             send_sems, recv_sems):
        my = lax.axis_index("i")

        barrier = pltpu.get_barrier_semaphore()
        for d in range(1, N_DEV):
            peer = lax.rem(my + d, N_DEV)
            pl.semaphore_signal(barrier, inc=1, device_id=(peer,),
                                device_id_type=pl.DeviceIdType.MESH)
        pl.semaphore_wait(barrier, N_DEV - 1)

        idx = idx_ref[:, :]
        eids = lax.broadcasted_iota(jnp.int32, (N_TOK, N_EXP), 1)
        onehot = idx == eids
        row = lax.broadcasted_iota(jnp.int32, (N_TOK, N_TOK), 0)
        col = lax.broadcasted_iota(jnp.int32, (N_TOK, N_TOK), 1)
        lower = (row > col).astype(jnp.float32)
        prior = jnp.dot(lower, onehot.astype(jnp.float32),
                        preferred_element_type=jnp.float32)
        keep = onehot & (prior < CAP)

        xin = x_ref[:, :]
        acc = jnp.zeros((N_TOK, D_OUT), jnp.float32)
        for k in range(E_LOC):
            e = my * E_LOC + k
            sel = jnp.where(keep & (eids == e), 1.0, 0.0)
            mask = jnp.sum(sel, axis=1, keepdims=True)
            xm = (xin * mask).astype(jnp.bfloat16)
            acc = acc + jnp.dot(xm, w_ref[k].astype(jnp.bfloat16),
                                preferred_element_type=jnp.float32)
        partial_ref[:, :] = acc

        comm_ref[0] = partial_ref[pl.ds(my * BLK, BLK), :]

        rdmas = []
        for d in range(1, N_DEV):
            tgt = lax.rem(my + d, N_DEV)
            rdma = pltpu.make_async_remote_copy(
                src_ref=partial_ref.at[pl.ds(tgt * BLK, BLK)],
                dst_ref=comm_ref.at[d],
                send_sem=send_sems.at[d],
                recv_sem=recv_sems.at[d],
                device_id=(tgt,),
                device_id_type=pl.DeviceIdType.MESH,
            )
            rdma.start()
            rdmas.append(rdma)

        total = comm_ref[0]
        for d in range(1, N_DEV):
            rdmas[d - 1].wait_recv()
            total = total + comm_ref[d]
        out_ref[:, :] = total

        for rdma in rdmas:
            rdma.wait_send()

    return pl.pallas_call(
        body,
        out_shape=jax.ShapeDtypeStruct((BLK, D_OUT), jnp.float32),
        in_specs=[
            pl.BlockSpec(memory_space=pltpu.VMEM),
            pl.BlockSpec(memory_space=pltpu.VMEM),
            pl.BlockSpec(memory_space=pltpu.VMEM),
        ],
        out_specs=pl.BlockSpec(memory_space=pltpu.VMEM),
        scratch_shapes=[
            pltpu.VMEM((N_TOK, D_OUT), jnp.float32),
            pltpu.VMEM((N_DEV, BLK, D_OUT), jnp.float32),
            pltpu.SemaphoreType.DMA((N_DEV,)),
            pltpu.SemaphoreType.DMA((N_DEV,)),
        ],
        compiler_params=pltpu.CompilerParams(collective_id=0),
    )(x, route_idx, expert_W)
